# baseline (speedup 1.0000x reference)
import functools

import jax
import jax.numpy as jnp
from jax import lax
from jax.experimental import pallas as pl
from jax.experimental.pallas import tpu as pltpu
from jax.experimental.pallas import tpu_sc as plsc

_BT = 128
_QB = 32
_SH = 5


@functools.lru_cache(maxsize=None)
def _make_gather(B, H, V, D):
    info = plsc.get_sparse_core_info()
    NC, NS, L = info.num_cores, info.num_subcores, info.num_lanes
    NW = NC * NS
    DT = D // 8
    n_bt = B // _BT
    uq = _BT // _QB
    urows = _QB * H
    n_units = n_bt * uq
    upw = n_units // NW
    nj = _QB // L
    assert H % (2 * _SH) == 0 and upw % 2 == 0
    mesh = plsc.VectorSubcoreMesh(core_axis_name="c", subcore_axis_name="s")

    @functools.partial(
        pl.kernel,
        mesh=mesh,
        out_type=jax.ShapeDtypeStruct((H, DT, n_bt, 8, _BT), jnp.float32),
        scratch_types=[
            pltpu.VMEM((2, urows), jnp.int32),
            pltpu.VMEM((2, urows, D), jnp.float32),
            pltpu.VMEM((_SH, DT, 8, _QB), jnp.float32),
            pltpu.VMEM((_SH, DT, 8, _QB), jnp.float32),
            pltpu.SemaphoreType.DMA,
            pltpu.SemaphoreType.DMA,
            pltpu.SemaphoreType.DMA,
            pltpu.SemaphoreType.DMA,
        ],
        compiler_params=pltpu.CompilerParams(
            use_tc_tiling_on_sc=False, needs_layout_passes=False),
    )
    def gather_kernel(table_hbm, idx_hbm, out_hbm, idxq, rows, slab_a,
                      slab_b, gsem0, gsem1, ssem_a, ssem_b):
        wid = lax.axis_index("s") * NC + lax.axis_index("c")
        u0 = wid * upw
        iota = lax.iota(jnp.int32, L)
        jvec = [(j * L + iota) * H for j in range(nj)]
        dvec = [jnp.full((L,), d, jnp.int32) for d in range(D)]
        slabs = (slab_a, slab_b)
        ssems = (ssem_a, ssem_b)
        gsems = (gsem0, gsem1)

        def start_gather(q, b):
            pltpu.sync_copy(idx_hbm.at[pl.ds(q * urows, urows)], idxq.at[b])
            pltpu.async_copy(table_hbm.at[idxq.at[b]], rows.at[b], gsems[b])

        def wait_gather(b):
            pltpu.make_async_copy(
                table_hbm.at[idxq.at[b]], rows.at[b], gsems[b]).wait()

        def transpose_unit(q, b):
            bt = q // uq
            bl0 = (q % uq) * _QB
            rbuf = rows.at[b]

            def out_slice(h0):
                return out_hbm.at[pl.ds(h0, _SH), :, bt, :,
                                  pl.ds(bl0, _QB)]

            def fill_slab(s, h0):
                for hh in range(_SH):
                    h = h0 + hh
                    lvec = [jv + h for jv in jvec]
                    for dt in range(DT):
                        vs = [
                            plsc.load_gather(
                                rbuf, [lvec[j], dvec[dt * 8 + ds]])
                            for ds in range(8)
                            for j in range(nj)
                        ]
                        for ds in range(8):
                            for j in range(nj):
                                slabs[s][hh, dt, ds, pl.ds(j * L, L)] = (
                                    vs[ds * nj + j])

            def p_body(p, carry):
                for s in range(2):
                    h0 = (2 * p + s) * _SH

                    @pl.when(p >= 1)
                    def _(s=s, h0=h0):
                        pltpu.make_async_copy(
                            slabs[s], out_slice(h0), ssems[s]).wait()

                    fill_slab(s, h0)
                    pltpu.async_copy(slabs[s], out_slice(h0), ssems[s])
                return carry

            lax.fori_loop(0, H // (2 * _SH), p_body, 0)
            for s in range(2):
                pltpu.make_async_copy(
                    slabs[s], out_slice(H - 2 * _SH + s * _SH),
                    ssems[s]).wait()

        start_gather(u0, 0)

        def u_body(k, carry):
            qa = u0 + 2 * k
            start_gather(qa + 1, 1)
            wait_gather(0)
            transpose_unit(qa, 0)

            @pl.when(k + 1 < upw // 2)
            def _():
                start_gather(qa + 2, 0)

            wait_gather(1)
            transpose_unit(qa + 1, 1)
            return carry

        lax.fori_loop(0, upw // 2, u_body, 0)

    return gather_kernel


def kernel(indices, table):
    B, H = indices.shape
    V, D = table.shape
    idx_flat = indices.reshape(B * H).astype(jnp.int32)
    o5 = _make_gather(B, H, V, D)(table, idx_flat)
    return o5.transpose(2, 4, 0, 1, 3).reshape(B, H, D)


# speedup vs baseline: 1.0065x; 1.0065x over previous
import functools

import jax
import jax.numpy as jnp
from jax import lax
from jax.experimental import pallas as pl
from jax.experimental.pallas import tpu as pltpu
from jax.experimental.pallas import tpu_sc as plsc

_BT = 128
_QB = 32


@functools.lru_cache(maxsize=None)
def _make_gather(B, H, V, D):
    info = plsc.get_sparse_core_info()
    NC, NS, L = info.num_cores, info.num_subcores, info.num_lanes
    NW = NC * NS
    DT = D // 8
    n_bt = B // _BT
    uq = _BT // _QB
    urows = _QB * H
    n_units = n_bt * uq
    upw = n_units // NW
    assert H % 2 == 0
    mesh = plsc.VectorSubcoreMesh(core_axis_name="c", subcore_axis_name="s")

    @functools.partial(
        pl.kernel,
        mesh=mesh,
        out_type=jax.ShapeDtypeStruct((H, DT, n_bt, 8, _BT), jnp.float32),
        scratch_types=[
            pltpu.VMEM((2, urows), jnp.int32),
            pltpu.VMEM((2, urows, D), jnp.float32),
            pltpu.VMEM((DT, 8, _QB), jnp.float32),
            pltpu.VMEM((DT, 8, _QB), jnp.float32),
            pltpu.SemaphoreType.DMA,
            pltpu.SemaphoreType.DMA,
            pltpu.SemaphoreType.DMA,
            pltpu.SemaphoreType.DMA,
        ],
        compiler_params=pltpu.CompilerParams(
            use_tc_tiling_on_sc=False, needs_layout_passes=False),
    )
    def gather_kernel(table_hbm, idx_hbm, out_hbm, idxq, rows, slab_a,
                      slab_b, gsem0, gsem1, ssem_a, ssem_b):
        wid = lax.axis_index("s") * NC + lax.axis_index("c")
        u0 = wid * upw
        iota = lax.iota(jnp.int32, L)
        jvec = [(j * L + iota) * H for j in range(_QB // L)]
        dvec = [jnp.full((L,), d, jnp.int32) for d in range(D)]
        slabs = (slab_a, slab_b)
        ssems = (ssem_a, ssem_b)
        gsems = (gsem0, gsem1)

        def start_gather(u, b):
            pltpu.sync_copy(
                idx_hbm.at[pl.ds((u0 + u) * urows, urows)], idxq.at[b])
            return pltpu.async_copy(table_hbm.at[idxq.at[b]], rows.at[b],
                                    gsems[b])

        def transpose_unit(u, b):
            bt = (u0 + u) // uq
            bl0 = ((u0 + u) % uq) * _QB
            rbuf = rows.at[b]

            def p_body(p, carry):
                for s in range(2):
                    h = 2 * p + s

                    @pl.when(p >= 1)
                    def _(s=s, h=h):
                        pltpu.make_async_copy(
                            slabs[s],
                            out_hbm.at[h, :, bt, :, pl.ds(bl0, _QB)],
                            ssems[s]).wait()

                    lvec = [jv + h for jv in jvec]
                    nj = _QB // L
                    for dt in range(DT):
                        vs = [
                            plsc.load_gather(
                                rbuf, [lvec[j], dvec[dt * 8 + ds]])
                            for ds in range(8)
                            for j in range(nj)
                        ]
                        for ds in range(8):
                            for j in range(nj):
                                slabs[s][dt, ds, pl.ds(j * L, L)] = (
                                    vs[ds * nj + j])
                    pltpu.async_copy(
                        slabs[s],
                        out_hbm.at[h, :, bt, :, pl.ds(bl0, _QB)],
                        ssems[s])
                return carry

            lax.fori_loop(0, H // 2, p_body, 0)
            for s in range(2):
                pltpu.make_async_copy(
                    slabs[s],
                    out_hbm.at[H - 2 + s, :, bt, :, pl.ds(bl0, _QB)],
                    ssems[s]).wait()

        gather = start_gather(0, 0)
        for u in range(upw):
            b = u & 1
            nxt = None
            if u + 1 < upw:
                nxt = start_gather(u + 1, 1 - b)
            gather.wait()
            transpose_unit(u, b)
            gather = nxt

    return gather_kernel


def kernel(indices, table):
    B, H = indices.shape
    V, D = table.shape
    idx_flat = indices.reshape(B * H).astype(jnp.int32)
    o5 = _make_gather(B, H, V, D)(table, idx_flat)
    return o5.transpose(2, 4, 0, 1, 3).reshape(B, H, D)
